# Initial kernel scaffold; baseline (speedup 1.0000x reference)
import jax
import jax.numpy as jnp
from jax.experimental import pallas as pl


def kernel(x, table, W_out, b_out):
    raise NotImplementedError("write your pallas kernel here")


# speedup vs baseline: 1.3414x; 1.3414x over previous
import functools

import jax
import jax.numpy as jnp
from jax import lax
from jax.experimental import pallas as pl
from jax.experimental.pallas import tpu as pltpu
from jax.experimental.pallas import tpu_sc as plsc

NC = 2
NS = 16
L = 16
NW = NC * NS

B = 16384
D = 128
OUT = 2
BPW = B // NW
CH = 128
NCHUNK = BPW // CH
DJ = D // L


def _sc_kernel(table_hbm, idx0_hbm, idx1_hbm, w_hbm, bpat_hbm, out_hbm,
               idx0_v, idx1_v, rows0_v, rows1_v, w_v, b_v, acc_v, logit_v,
               sem0, sem1):
    wid = lax.axis_index("s") * NC + lax.axis_index("c")
    base = wid * BPW

    pltpu.sync_copy(w_hbm, w_v)
    pltpu.sync_copy(bpat_hbm, b_v)

    w0 = [w_v[0, pl.ds(j * L, L)] for j in range(DJ)]
    w1 = [w_v[1, pl.ds(j * L, L)] for j in range(DJ)]
    bvec = b_v[...]
    lanes = lax.iota(jnp.int32, L)
    tr_idx = [lanes * L + l for l in range(L)]

    for c in range(NCHUNK):
        off = base + c * CH
        pltpu.sync_copy(idx0_hbm.at[pl.ds(off, CH)], idx0_v)
        pltpu.sync_copy(idx1_hbm.at[pl.ds(off, CH)], idx1_v)
        cp0 = pltpu.async_copy(table_hbm.at[idx0_v], rows0_v, sem0)
        cp1 = pltpu.async_copy(table_hbm.at[idx1_v], rows1_v, sem1)
        cp0.wait()
        cp1.wait()

        def grp_body(g, _, c=c):
            for i in range(8):
                b = g * 8 + i
                acc0 = jnp.zeros((L,), jnp.float32)
                acc1 = jnp.zeros((L,), jnp.float32)
                for j in range(DJ):
                    e0 = rows0_v[b, pl.ds(j * L, L)]
                    e1 = rows1_v[b, pl.ds(j * L, L)]
                    h = e0 * e1
                    acc0 = acc0 + h * w0[j]
                    acc1 = acc1 + h * w1[j]
                acc_v[pl.ds(2 * i * L, L)] = acc0
                acc_v[pl.ds((2 * i + 1) * L, L)] = acc1
            tot = plsc.load_gather(acc_v, [tr_idx[0]])
            for l in range(1, L):
                tot = tot + plsc.load_gather(acc_v, [tr_idx[l]])
            z = tot + bvec
            logit_v[pl.ds(c * CH * OUT + g * L, L)] = 1.0 / (1.0 + jnp.exp(-z))
            return _

        lax.fori_loop(0, CH // 8, grp_body, None)

    pltpu.sync_copy(logit_v, out_hbm.at[pl.ds(base * OUT, BPW * OUT)])


@jax.jit
def _run(table, idx0, idx1, W_out, b_out):
    mesh = plsc.VectorSubcoreMesh(core_axis_name="c", subcore_axis_name="s")
    kern = functools.partial(
        pl.kernel,
        out_type=jax.ShapeDtypeStruct((B * OUT,), jnp.float32),
        mesh=mesh,
        compiler_params=pltpu.CompilerParams(needs_layout_passes=False),
        scratch_types=[
            pltpu.VMEM((CH,), jnp.int32),
            pltpu.VMEM((CH,), jnp.int32),
            pltpu.VMEM((CH, D), jnp.float32),
            pltpu.VMEM((CH, D), jnp.float32),
            pltpu.VMEM((OUT, D), jnp.float32),
            pltpu.VMEM((L,), jnp.float32),
            pltpu.VMEM((L * L,), jnp.float32),
            pltpu.VMEM((BPW * OUT,), jnp.float32),
            pltpu.SemaphoreType.DMA,
            pltpu.SemaphoreType.DMA,
        ],
    )(_sc_kernel)
    bpat = jnp.tile(b_out, L // OUT)
    flat = kern(table, idx0, idx1, W_out, bpat)
    return flat.reshape(B, OUT)


def kernel(x, table, W_out, b_out):
    idx0 = x[:, 0].astype(jnp.int32)
    idx1 = x[:, 1].astype(jnp.int32)
    return _run(table, idx0, idx1, W_out, b_out)


# speedup vs baseline: 1.5321x; 1.1421x over previous
import functools

import jax
import jax.numpy as jnp
from jax import lax
from jax.experimental import pallas as pl
from jax.experimental.pallas import tpu as pltpu
from jax.experimental.pallas import tpu_sc as plsc

NC = 2
NS = 16
L = 16
NW = NC * NS

B = 16384
D = 128
OUT = 2
BPW = B // NW
CH = 128
NCHUNK = BPW // CH
DJ = D // L


def _sc_kernel(table_hbm, idx0_hbm, idx1_hbm, w_hbm, bpat_hbm, out_hbm,
               idx0_v, idx1_v, rows0_v, rows1_v, w_v, b_v, acc_v, logit_v,
               sem0, sem1):
    wid = lax.axis_index("s") * NC + lax.axis_index("c")
    base = wid * BPW

    pltpu.sync_copy(w_hbm, w_v)
    pltpu.sync_copy(bpat_hbm, b_v)

    w0 = [w_v[0, pl.ds(j * L, L)] for j in range(DJ)]
    w1 = [w_v[1, pl.ds(j * L, L)] for j in range(DJ)]
    bvec = b_v[...]
    lanes = lax.iota(jnp.int32, L)
    tr_idx = [lanes * L + l for l in range(L)]
    row_off = lax.shift_right_logical(lanes, 1)
    col_idx = lanes & 1

    for c in range(NCHUNK):
        off = base + c * CH
        pltpu.sync_copy(idx0_hbm.at[pl.ds(off, CH)], idx0_v)
        pltpu.sync_copy(idx1_hbm.at[pl.ds(off, CH)], idx1_v)
        cp0 = pltpu.async_copy(table_hbm.at[idx0_v], rows0_v, sem0)
        cp1 = pltpu.async_copy(table_hbm.at[idx1_v], rows1_v, sem1)
        cp0.wait()
        cp1.wait()

        def grp_body(g, _, c=c):
            for i in range(8):
                b = g * 8 + i
                acc0 = jnp.zeros((L,), jnp.float32)
                acc1 = jnp.zeros((L,), jnp.float32)
                for j in range(DJ):
                    e0 = rows0_v[b, pl.ds(j * L, L)]
                    e1 = rows1_v[b, pl.ds(j * L, L)]
                    h = e0 * e1
                    acc0 = acc0 + h * w0[j]
                    acc1 = acc1 + h * w1[j]
                acc_v[pl.ds(2 * i * L, L)] = acc0
                acc_v[pl.ds((2 * i + 1) * L, L)] = acc1
            tot = plsc.load_gather(acc_v, [tr_idx[0]])
            for l in range(1, L):
                tot = tot + plsc.load_gather(acc_v, [tr_idx[l]])
            z = tot + bvec
            sig = 1.0 / (1.0 + jnp.exp(-z))
            rows = (c * CH + g * 8) + row_off
            plsc.store_scatter(logit_v, [rows, col_idx], sig)
            return _

        lax.fori_loop(0, CH // 8, grp_body, None)

    pltpu.sync_copy(logit_v, out_hbm.at[pl.ds(base, BPW), :])


@jax.jit
def _run(table, idx0, idx1, W_out, b_out):
    mesh = plsc.VectorSubcoreMesh(core_axis_name="c", subcore_axis_name="s")
    kern = functools.partial(
        pl.kernel,
        out_type=jax.ShapeDtypeStruct((B, OUT), jnp.float32),
        mesh=mesh,
        compiler_params=pltpu.CompilerParams(needs_layout_passes=False),
        scratch_types=[
            pltpu.VMEM((CH,), jnp.int32),
            pltpu.VMEM((CH,), jnp.int32),
            pltpu.VMEM((CH, D), jnp.float32),
            pltpu.VMEM((CH, D), jnp.float32),
            pltpu.VMEM((OUT, D), jnp.float32),
            pltpu.VMEM((L,), jnp.float32),
            pltpu.VMEM((L * L,), jnp.float32),
            pltpu.VMEM((BPW, OUT), jnp.float32),
            pltpu.SemaphoreType.DMA,
            pltpu.SemaphoreType.DMA,
        ],
    )(_sc_kernel)
    bpat = jnp.tile(b_out, L // OUT)
    return kern(table, idx0, idx1, W_out, bpat)


def kernel(x, table, W_out, b_out):
    idx0 = x[:, 0].astype(jnp.int32)
    idx1 = x[:, 1].astype(jnp.int32)
    return _run(table, idx0, idx1, W_out, b_out)
